# baseline (speedup 1.0000x reference)
import functools

import jax
import jax.numpy as jnp
from jax import lax
from jax.experimental import pallas as pl
from jax.experimental.pallas import tpu as pltpu
from jax.experimental.pallas import tpu_sc as plsc

N_USERS = 1000000
N_HORIZONS = 16
BATCH = 16384

NUM_CORES = 2
NUM_SUBCORES = 16
NUM_WORKERS = NUM_CORES * NUM_SUBCORES
ROWS_PER_WORKER = BATCH // NUM_WORKERS


def _sc_gather_kernel(scale_hbm, bias_hbm, idx_hbm, scale_out, bias_out,
                      idx_v, s_v, b_v, sem_s, sem_b):
    wid = lax.axis_index("s") * NUM_CORES + lax.axis_index("c")
    base = wid * ROWS_PER_WORKER
    pltpu.sync_copy(idx_hbm.at[pl.ds(base, ROWS_PER_WORKER)], idx_v)
    cp_s = pltpu.async_copy(scale_hbm.at[idx_v], s_v, sem_s)
    cp_b = pltpu.async_copy(bias_hbm.at[idx_v], b_v, sem_b)
    cp_s.wait()
    cp_b.wait()
    pltpu.sync_copy(s_v, scale_out.at[pl.ds(base, ROWS_PER_WORKER)])
    pltpu.sync_copy(b_v, bias_out.at[pl.ds(base, ROWS_PER_WORKER)])


def _sc_gather(scale_table, bias_table, idx):
    mesh = plsc.VectorSubcoreMesh(core_axis_name="c", subcore_axis_name="s")
    row = jax.ShapeDtypeStruct((BATCH, N_HORIZONS), jnp.float32)
    kern = pl.kernel(
        _sc_gather_kernel,
        mesh=mesh,
        out_type=(row, row),
        scratch_types=[
            pltpu.VMEM((ROWS_PER_WORKER,), jnp.int32),
            pltpu.VMEM((ROWS_PER_WORKER, N_HORIZONS), jnp.float32),
            pltpu.VMEM((ROWS_PER_WORKER, N_HORIZONS), jnp.float32),
            pltpu.SemaphoreType.DMA,
            pltpu.SemaphoreType.DMA,
        ],
        compiler_params=pltpu.CompilerParams(use_tc_tiling_on_sc=False),
    )
    return kern(scale_table, bias_table, idx)


def _tc_math_kernel(p_ref, s_ref, b_ref, o_ref):
    eps = 1e-07
    p = jnp.clip(p_ref[...], eps, 1.0 - eps)
    logits = jnp.log(p / (1.0 - p))
    o_ref[...] = jax.nn.sigmoid(logits * s_ref[...] + b_ref[...])


def _tc_math(p2, s2, b2):
    return pl.pallas_call(
        _tc_math_kernel,
        out_shape=jax.ShapeDtypeStruct(p2.shape, jnp.float32),
    )(p2, s2, b2)


@jax.jit
def kernel(probs, user_ids, scale_table, bias_table):
    idx = user_ids.astype(jnp.int32)
    scale_g, bias_g = _sc_gather(scale_table, bias_table, idx)
    flat = (BATCH * N_HORIZONS // 128, 128)
    out = _tc_math(probs.reshape(flat), scale_g.reshape(flat),
                   bias_g.reshape(flat))
    return out.reshape(BATCH, N_HORIZONS)
